# Initial kernel scaffold; baseline (speedup 1.0000x reference)
import jax
import jax.numpy as jnp
from jax.experimental import pallas as pl


def kernel(inputs):
    raise NotImplementedError("write your pallas kernel here")


# speedup vs baseline: 3.3775x; 3.3775x over previous
import functools

import jax
import jax.numpy as jnp
from jax import lax
from jax.experimental import pallas as pl
from jax.experimental.pallas import tpu as pltpu
from jax.experimental.pallas import tpu_sc as plsc

MS = 512
B = 128
N = MS * (MS + 1) // 2
RB = 64
NBLK = MS // RB
L = 16
NJ = MS // L


def _off(r: int) -> int:
    return MS * r - r * (r - 1) // 2


_SPAN0 = [_off(k * RB) for k in range(NBLK)]
_SPANLEN = [_off((k + 1) * RB) - _off(k * RB) for k in range(NBLK)]
_MAXSPAN = max(_SPANLEN)

_info = plsc.get_sparse_core_info()
_NC, _NS = _info.num_cores, _info.num_subcores
_NW = _NC * _NS
_BPW = B // _NW


def kernel(inputs):
    @functools.partial(
        pl.kernel,
        mesh=plsc.VectorSubcoreMesh(core_axis_name="c", subcore_axis_name="s"),
        out_type=jax.ShapeDtypeStruct((B * MS * MS,), jnp.float32),
        scratch_types=[
            pltpu.VMEM((_MAXSPAN,), jnp.float32),
            pltpu.VMEM((RB * MS,), jnp.float32),
        ],
    )
    def sc_kernel(in_hbm, out_hbm, in_v, out_v):
        wid = lax.axis_index("s") * _NC + lax.axis_index("c")
        lane = jnp.arange(L, dtype=jnp.int32)
        zeros = jnp.zeros((L,), jnp.float32)

        def per_batch(i, carry):
            b = wid * _BPW + i
            for k in range(NBLK):
                span0 = _SPAN0[k]
                slen = _SPANLEN[k]
                pltpu.sync_copy(
                    in_hbm.at[pl.ds(b * N + span0, slen)], in_v.at[pl.ds(0, slen)]
                )

                def per_row(rl, c2, k=k, span0=span0):
                    r = k * RB + rl
                    off_r = MS * r - (r * (r - 1)) // 2
                    bl = off_r - r - span0
                    for j in range(NJ):
                        c0 = j * L

                        @pl.when(c0 + (L - 1) >= r)
                        def _(j=j, c0=c0, bl=bl, r=r, rl=rl):
                            v = in_v[pl.ds(bl + c0, L)]
                            m = (lane + c0) >= r
                            out_v[pl.ds(rl * MS + c0, L)] = jnp.where(m, v, 0.0)

                        @pl.when(c0 + (L - 1) < r)
                        def _(c0=c0, rl=rl):
                            out_v[pl.ds(rl * MS + c0, L)] = zeros

                    return c2

                lax.fori_loop(0, RB, per_row, 0)
                pltpu.sync_copy(
                    out_v, out_hbm.at[pl.ds(b * (MS * MS) + k * RB * MS, RB * MS)]
                )
            return carry

        lax.fori_loop(0, _BPW, per_batch, 0)

    out_flat = sc_kernel(inputs.reshape(-1))
    return out_flat.reshape(B, MS, MS)
